# baseline (speedup 1.0000x reference)
import functools
import math

import jax
import jax.numpy as jnp
from jax import lax
from jax.experimental import pallas as pl
from jax.experimental.pallas import tpu as pltpu
from jax.experimental.pallas import tpu_sc as plsc

_N = 10000
_D = 128
_E = 320000
_NE = 5000
_NT = 4

_NC = 2
_NS = 16
_NW = _NC * _NS
_L = 16

_NE_PAD = 5120
_N_PAD = 10112
_C = 80
_PER_W = _E // _NW
_NCH = _PER_W // _C
_CB = 40
_NCHB = _PER_W // _CB

_F32 = jnp.float32
_I32 = jnp.int32


def _prep_body(x_ref, tid_ref, wpoi_ref, wcat_ref, wreg_ref,
               wq0, wk0, wq1, wk1, wq2, wk2, wq3, wk3,
               h_ref, xm0_ref, xm1_ref, xm2_ref, xm3_ref):
    xb = x_ref[...]
    hp = jnp.dot(xb, wpoi_ref[...].T, preferred_element_type=_F32)
    hc = jnp.dot(xb, wcat_ref[...].T, preferred_element_type=_F32)
    hr = jnp.dot(xb, wreg_ref[...].T, preferred_element_type=_F32)
    tid = tid_ref[...]
    h_ref[...] = jnp.where(tid == 0.0, hp, jnp.where(tid == 1.0, hc, hr))
    scale = 1.0 / math.sqrt(_D)
    for wq, wk, xm in ((wq0, wk0, xm0_ref), (wq1, wk1, xm1_ref),
                       (wq2, wk2, xm2_ref), (wq3, wk3, xm3_ref)):
        m = jnp.dot(wq[...].T, wk[...], preferred_element_type=_F32)
        xm[...] = jnp.dot(xb, m, preferred_element_type=_F32) * scale


def _tc_prep(x, tidf, wpoi, wcat, wreg, wqs, wks):
    blk = 1000
    grid = (_N // blk,)
    row_spec = pl.BlockSpec((blk, _D), lambda i: (i, 0))
    w_spec = pl.BlockSpec((_D, _D), lambda i: (0, 0))
    out = pl.pallas_call(
        _prep_body,
        grid=grid,
        in_specs=[row_spec, pl.BlockSpec((blk, 1), lambda i: (i, 0))]
        + [w_spec] * 11,
        out_specs=[row_spec] * 5,
        out_shape=[jax.ShapeDtypeStruct((_N, _D), _F32)] * 5,
    )(x, tidf, wpoi, wcat, wreg,
      wqs[0], wks[0], wqs[1], wks[1], wqs[2], wks[2], wqs[3], wks[3])
    return out[0], out[1:]


def _edge_body(esp_ref, cnt_ref, ew_ref, we_ref,
               ee0_ref, ee1_ref, ee2_ref, ee3_ref):
    outs = (ee0_ref, ee1_ref, ee2_ref, ee3_ref)
    for t in range(_NT):
        es = esp_ref[t, 0] + esp_ref[t, 1]
        cnt = cnt_ref[t, 0] + cnt_ref[t, 1]
        emb = es / jnp.clip(cnt, 1.0, None)[:, None]
        emb = jnp.dot(emb, we_ref[t].T, preferred_element_type=_F32)
        outs[t][...] = emb * ew_ref[t][:, None]


def _tc_edge(esp, cntp, ew_stack, we_stack):
    return pl.pallas_call(
        _edge_body,
        out_shape=[jax.ShapeDtypeStruct((_NE, _D), _F32)] * 4,
    )(esp, cntp, ew_stack, we_stack)


def _final_body(x_ref, num_ref, den_ref, wv_ref, g_ref, b_ref, o_ref):
    xb = x_ref[...]
    acc = jnp.zeros_like(xb)
    for t in range(_NT):
        nsum = num_ref[t, 0] + num_ref[t, 1]
        dsum = jnp.sum(den_ref[t], axis=0)[:, 0]
        tot = nsum / (dsum + 1e-12)[:, None]
        acc = acc + jnp.dot(tot, wv_ref[t].T, preferred_element_type=_F32)
    out = xb + acc * (1.0 / _NT)
    mean = jnp.mean(out, axis=-1, keepdims=True)
    var = jnp.mean((out - mean) ** 2, axis=-1, keepdims=True)
    o_ref[...] = (out - mean) / jnp.sqrt(var + 1e-5) * g_ref[...] + b_ref[...]


def _tc_final(x, nump, denp4, wv_stack, gamma, beta):
    blk = 1000
    grid = (_N // blk,)
    return pl.pallas_call(
        _final_body,
        grid=grid,
        in_specs=[
            pl.BlockSpec((blk, _D), lambda i: (i, 0)),
            pl.BlockSpec((_NT, _NC, blk, _D), lambda i: (0, 0, i, 0)),
            pl.BlockSpec((_NT, _NC, blk, 1), lambda i: (0, 0, i, 0)),
            pl.BlockSpec((_NT, _D, _D), lambda i: (0, 0, 0)),
            pl.BlockSpec((1, _D), lambda i: (0, 0)),
            pl.BlockSpec((1, _D), lambda i: (0, 0)),
        ],
        out_specs=pl.BlockSpec((blk, _D), lambda i: (i, 0)),
        out_shape=jax.ShapeDtypeStruct((_N, _D), _F32),
    )(x, nump, denp4, wv_stack, gamma, beta)


def _lane_shuffle(v, perm):
    dnums = lax.GatherDimensionNumbers(
        offset_dims=(), collapsed_slice_dims=(0,), start_index_map=(0,))
    return lax.gather(v, perm[:, None], dnums, (1,),
                      mode=lax.GatherScatterMode.PROMISE_IN_BOUNDS)


def _sc_mesh():
    return plsc.VectorSubcoreMesh(core_axis_name="c", subcore_axis_name="s")


def _zero_1d(ref, n):
    def zb(i, carry):
        ref[pl.ds(i * _L, _L)] = jnp.zeros((_L,), _F32)
        return carry
    lax.fori_loop(0, n // _L, zb, 0)


_NB = 4


def _segsum_body(h, nid0, nid1, nid2, nid3, eid0, eid1, eid2, eid3,
                 z2d, es_out, cnt_out,
                 nid_r0, nid_r1, nid_r2, nid_r3,
                 eid_r0, eid_r1, eid_r2, eid_r3,
                 rows_r0, rows_r1, rows_r2, rows_r3,
                 ones_b, zb_b, stage_b, es_acc, cnt_acc,
                 s_in, s_ie, g0, g1, sr0, sr1, sr2, sr3,
                 so0, so1, so2, so3):
    c = lax.axis_index("c")
    s = lax.axis_index("s")
    wid = s * _NC + c
    nids = (nid0, nid1, nid2, nid3)
    eids = (eid0, eid1, eid2, eid3)
    nid_r = (nid_r0, nid_r1, nid_r2, nid_r3)
    eid_r = (eid_r0, eid_r1, eid_r2, eid_r3)
    rows_r = (rows_r0, rows_r1, rows_r2, rows_r3)
    g_sem = (g0, g1)
    sr = (sr0, sr1, sr2, sr3)
    so = (so0, so1, so2, so3)
    for j in range(_C // _L):
        ones_b[pl.ds(j * _L, _L)] = jnp.ones((_L,), _F32)
    _zero_1d(zb_b, _NE_PAD)

    def idx_load(t, j, b):
        base = wid * _PER_W + j * _C
        d1 = pltpu.async_copy(nids[t].at[pl.ds(base, _C)], nid_r[b], s_in)
        d2 = pltpu.async_copy(eids[t].at[pl.ds(base, _C)], eid_r[b], s_ie)
        d1.wait()
        d2.wait()

    def gather_start(b, b2):
        pltpu.async_copy(h.at[nid_r[b]], rows_r[b], g_sem[b2])

    def gather_wait(b, b2):
        pltpu.make_async_copy(h.at[nid_r[b]], rows_r[b], g_sem[b2]).wait()

    def scatter_start(b):
        pltpu.async_copy(rows_r[b], es_acc.at[eid_r[b]], sr[b], add=True)
        pltpu.async_copy(ones_b, cnt_acc.at[eid_r[b]], so[b], add=True)

    def scatter_wait(b):
        pltpu.make_async_copy(rows_r[b], es_acc.at[eid_r[b]], sr[b]).wait()
        pltpu.make_async_copy(ones_b, cnt_acc.at[eid_r[b]], so[b]).wait()

    for t in range(_NT):
        @pl.when(s == 0)
        def _zero():
            pltpu.sync_copy(z2d.at[pl.ds(0, _NE)], es_acc)
            pltpu.sync_copy(zb_b, cnt_acc)
        plsc.subcore_barrier()

        idx_load(t, 0, 0)
        gather_start(0, 0)

        def superstep(sidx, carry):
            for b in range(_NB):
                j = sidx * _NB + b
                bq = (b + 1) % _NB

                @pl.when(j >= 3)
                def _drain():
                    scatter_wait(bq)
                idx_load(t, j + 1, bq)
                gather_start(bq, (b + 1) % 2)
                gather_wait(b, b % 2)
                scatter_start(b)
            return carry
        lax.fori_loop(0, (_NCH - 1) // _NB, superstep, 0)
        scatter_wait(1)
        gather_wait(0, 0)
        scatter_start(0)
        for b in (2, 3, 0):
            scatter_wait(b)
        plsc.subcore_barrier()

        @pl.when(s == 0)
        def _out():
            pltpu.sync_copy(es_acc, es_out.at[t, c])
            pltpu.sync_copy(cnt_acc.at[pl.ds(0, _NE)], stage_b)
            pltpu.sync_copy(stage_b, cnt_out.at[t, c])
        plsc.subcore_barrier()


def _sc_segsum(h_node, nids, eids, z2d):
    fn = pl.kernel(
        _segsum_body,
        out_type=(jax.ShapeDtypeStruct((_NT, _NC, _NE, _D), _F32),
                  jax.ShapeDtypeStruct((_NT, _NC, _NE), _F32)),
        mesh=_sc_mesh(),
        scratch_types=(
            [pltpu.VMEM((_C,), _I32)] * 8
            + [pltpu.VMEM((_C, _D), _F32)] * 4
            + [pltpu.VMEM((_C,), _F32),
               pltpu.VMEM((_NE_PAD,), _F32),
               pltpu.VMEM((_NE,), _F32),
               pltpu.VMEM_SHARED((_NE, _D), _F32),
               pltpu.VMEM_SHARED((_NE_PAD,), _F32)]
            + [pltpu.SemaphoreType.DMA] * 12
        ),
    )
    return fn(h_node, *nids, *eids, z2d)


_NBI = 8


def _attn_body(xm0, xm1, xm2, xm3, ee0, ee1, ee2, ee3,
               nid0, nid1, nid2, nid3, eid0, eid1, eid2, eid3,
               z2d, num_out, den_out,
               nid_r0, nid_r1, nid_r2, nid_r3,
               nid_r4, nid_r5, nid_r6, nid_r7,
               eid_r0, eid_r1, eid_r2, eid_r3,
               eid_r4, eid_r5, eid_r6, eid_r7,
               xr_r0, xr_r1,
               er_r0, er_r1, er_r2, er_r3,
               ex_r0, ex_r1, ex_r2, ex_r3,
               buf_b, num_acc, den_acc,
               si0, si1, se0, se1, gx0, gx1, ge0, ge1,
               sn0, sn1, sn2, sn3, sd0, sd1, sd2, sd3):
    c = lax.axis_index("c")
    s = lax.axis_index("s")
    wid = s * _NC + c
    xms = (xm0, xm1, xm2, xm3)
    ees = (ee0, ee1, ee2, ee3)
    nids = (nid0, nid1, nid2, nid3)
    eids = (eid0, eid1, eid2, eid3)
    nid_r = (nid_r0, nid_r1, nid_r2, nid_r3,
             nid_r4, nid_r5, nid_r6, nid_r7)
    eid_r = (eid_r0, eid_r1, eid_r2, eid_r3,
             eid_r4, eid_r5, eid_r6, eid_r7)
    xr_r = (xr_r0, xr_r1)
    er_r = (er_r0, er_r1, er_r2, er_r3)
    ex_r = (ex_r0, ex_r1, ex_r2, ex_r3)
    si_sem = (si0, si1)
    se_sem = (se0, se1)
    gx_sem = (gx0, gx1)
    ge_sem = (ge0, ge1)
    sn = (sn0, sn1, sn2, sn3)
    sd = (sd0, sd1, sd2, sd3)
    lanes = lax.iota(_I32, _L)
    lane0 = lanes == 0

    def idx_start(t, j, m):
        b, p = m, m % 2
        base = wid * _PER_W + j * _CB
        pltpu.async_copy(nids[t].at[pl.ds(base, _CB)], nid_r[b], si_sem[p])
        pltpu.async_copy(eids[t].at[pl.ds(base, _CB)], eid_r[b], se_sem[p])

    def idx_wait(t, j, m):
        b, p = m, m % 2
        base = wid * _PER_W + j * _CB
        pltpu.make_async_copy(nids[t].at[pl.ds(base, _CB)], nid_r[b],
                              si_sem[p]).wait()
        pltpu.make_async_copy(eids[t].at[pl.ds(base, _CB)], eid_r[b],
                              se_sem[p]).wait()

    def gather_start(t, m):
        b, b4, b2 = m, m % _NB, m % 2
        pltpu.async_copy(xms[t].at[nid_r[b]], xr_r[b2], gx_sem[b2])
        pltpu.async_copy(ees[t].at[eid_r[b]], er_r[b4], ge_sem[b2])

    def gather_wait(t, m):
        b, b4, b2 = m, m % _NB, m % 2
        pltpu.make_async_copy(xms[t].at[nid_r[b]], xr_r[b2],
                              gx_sem[b2]).wait()
        pltpu.make_async_copy(ees[t].at[eid_r[b]], er_r[b4],
                              ge_sem[b2]).wait()

    def scatter_start(m):
        b, b4 = m, m % _NB
        pltpu.async_copy(er_r[b4], num_acc.at[nid_r[b]], sn[b4], add=True)
        pltpu.async_copy(ex_r[b4].at[pl.ds(0, _CB)], den_acc.at[nid_r[b]],
                         sd[b4], add=True)

    def scatter_wait(m):
        b, b4 = m, m % _NB
        pltpu.make_async_copy(er_r[b4], num_acc.at[nid_r[b]],
                              sn[b4]).wait()
        pltpu.make_async_copy(ex_r[b4].at[pl.ds(0, _CB)],
                              den_acc.at[nid_r[b]], sd[b4]).wait()

    def compute(m):
        xr_b = xr_r[m % 2]
        er_b = er_r[m % _NB]
        ex_b = ex_r[m % _NB]

        def inc(i, icarry):
            evs = []
            ps = []
            for d in range(_D // _L):
                xv = xr_b[i, pl.ds(d * _L, _L)]
                ev = er_b[i, pl.ds(d * _L, _L)]
                evs.append(ev)
                ps.append(xv * ev)
            while len(ps) > 1:
                ps = [ps[k] + ps[k + 1] for k in range(0, len(ps), 2)]
            sc = ps[0]
            for k in (8, 4, 2, 1):
                perm = jnp.bitwise_xor(lanes, k)
                sc = sc + _lane_shuffle(sc, perm)
            exv = jnp.exp(sc)
            for d in range(_D // _L):
                er_b[i, pl.ds(d * _L, _L)] = evs[d] * exv
            ex_b[pl.ds(i, _L)] = exv
            return icarry
        lax.fori_loop(0, _CB, inc, 0)

    for t in range(_NT):
        @pl.when(s == 0)
        def _zero():
            _zero_1d(buf_b, _N - _L)
            buf_b[pl.ds(_N - _L, _L)] = jnp.zeros((_L,), _F32)
            pltpu.sync_copy(z2d, num_acc)
            pltpu.sync_copy(buf_b, den_acc)
        plsc.subcore_barrier()

        idx_start(t, 0, 0)
        idx_wait(t, 0, 0)
        gather_start(t, 0)
        idx_start(t, 1, 1)
        n_ss = (_NCHB + 3 + _NBI - 1) // _NBI

        def superstep(sidx, carry):
            for b in range(_NBI):
                j = sidx * _NBI + b

                @pl.when(jnp.logical_and(j >= 3, j - 3 < _NCHB))
                def _drain():
                    scatter_wait((b + 5) % _NBI)

                @pl.when(j + 1 < _NCHB)
                def _pref1():
                    idx_wait(t, j + 1, (b + 1) % _NBI)
                    gather_start(t, (b + 1) % _NBI)

                @pl.when(j + 2 < _NCHB)
                def _pref2():
                    idx_start(t, j + 2, (b + 2) % _NBI)

                @pl.when(j < _NCHB)
                def _work():
                    gather_wait(t, b)
                    compute(b)
                    scatter_start(b)
            return carry
        lax.fori_loop(0, n_ss, superstep, 0)
        plsc.subcore_barrier()

        @pl.when(s == 0)
        def _out():
            pltpu.sync_copy(num_acc, num_out.at[t, c])
            pltpu.sync_copy(den_acc, buf_b)
            pltpu.sync_copy(buf_b, den_out.at[t, c])
        plsc.subcore_barrier()


def _sc_attn(xms, ees, nids, eids, z2d):
    fn = pl.kernel(
        _attn_body,
        out_type=(jax.ShapeDtypeStruct((_NT, _NC, _N, _D), _F32),
                  jax.ShapeDtypeStruct((_NT, _NC, _N), _F32)),
        mesh=_sc_mesh(),
        scratch_types=(
            [pltpu.VMEM((_CB,), _I32)] * 16
            + [pltpu.VMEM((_CB, _D), _F32)] * 6
            + [pltpu.VMEM((_CB + _L,), _F32)] * 4
            + [pltpu.VMEM((_N,), _F32),
               pltpu.VMEM_SHARED((_N, _D), _F32),
               pltpu.VMEM_SHARED((_N,), _F32)]
            + [pltpu.SemaphoreType.DMA] * 16
        ),
    )
    return fn(*xms, *ees, *nids, *eids, z2d)


def kernel(x, node_type_ids,
           node_ids_func, edge_ids_func, edge_weight_func,
           node_ids_region, edge_ids_region, edge_weight_region,
           node_ids_geo, edge_ids_geo, edge_weight_geo,
           node_ids_mob, edge_ids_mob, edge_weight_mob,
           W_poi, W_cat, W_reg,
           W_edge_func, Wq_func, Wk_func, Wv_func,
           W_edge_region, Wq_region, Wk_region, Wv_region,
           W_edge_geo, Wq_geo, Wk_geo, Wv_geo,
           W_edge_mob, Wq_mob, Wk_mob, Wv_mob,
           ln_gamma, ln_beta):
    nids = (node_ids_func, node_ids_region, node_ids_geo, node_ids_mob)
    eids = (edge_ids_func, edge_ids_region, edge_ids_geo, edge_ids_mob)
    ews = (edge_weight_func, edge_weight_region, edge_weight_geo,
           edge_weight_mob)
    wqs = (Wq_func, Wq_region, Wq_geo, Wq_mob)
    wks = (Wk_func, Wk_region, Wk_geo, Wk_mob)
    wvs = (Wv_func, Wv_region, Wv_geo, Wv_mob)
    wes = (W_edge_func, W_edge_region, W_edge_geo, W_edge_mob)

    tidf = node_type_ids.astype(_F32).reshape(_N, 1)
    z2d = jnp.zeros((_N, _D), _F32)

    h_node, xms = _tc_prep(x, tidf, W_poi, W_cat, W_reg, wqs, wks)
    esp, cntp = _sc_segsum(h_node, nids, eids, z2d)
    ees = _tc_edge(esp, cntp, jnp.stack(ews), jnp.stack(wes))
    nump, denp = _sc_attn(xms, ees, nids, eids, z2d)
    out = _tc_final(x, nump, denp.reshape(_NT, _NC, _N, 1),
                    jnp.stack(wvs), ln_gamma.reshape(1, _D),
                    ln_beta.reshape(1, _D))
    return out


# speedup vs baseline: 1.0943x; 1.0943x over previous
import functools
import math

import jax
import jax.numpy as jnp
from jax import lax
from jax.experimental import pallas as pl
from jax.experimental.pallas import tpu as pltpu
from jax.experimental.pallas import tpu_sc as plsc

_N = 10000
_D = 128
_E = 320000
_NE = 5000
_NT = 4

_NC = 2
_NS = 16
_NW = _NC * _NS
_L = 16

_NE_PAD = 5120
_N_PAD = 10112
_C = 80
_PER_W = _E // _NW
_NCH = _PER_W // _C
_CB = 40
_NCHB = _PER_W // _CB

_F32 = jnp.float32
_I32 = jnp.int32


def _prep_body(x_ref, tid_ref, wpoi_ref, wcat_ref, wreg_ref,
               wq0, wk0, wq1, wk1, wq2, wk2, wq3, wk3,
               h_ref, xm0_ref, xm1_ref, xm2_ref, xm3_ref):
    xb = x_ref[...]
    hp = jnp.dot(xb, wpoi_ref[...].T, preferred_element_type=_F32)
    hc = jnp.dot(xb, wcat_ref[...].T, preferred_element_type=_F32)
    hr = jnp.dot(xb, wreg_ref[...].T, preferred_element_type=_F32)
    tid = tid_ref[...]
    h_ref[...] = jnp.where(tid == 0.0, hp, jnp.where(tid == 1.0, hc, hr))
    scale = 1.0 / math.sqrt(_D)
    for wq, wk, xm in ((wq0, wk0, xm0_ref), (wq1, wk1, xm1_ref),
                       (wq2, wk2, xm2_ref), (wq3, wk3, xm3_ref)):
        m = jnp.dot(wq[...].T, wk[...], preferred_element_type=_F32)
        xm[...] = jnp.dot(xb, m, preferred_element_type=_F32) * scale


def _tc_prep(x, tidf, wpoi, wcat, wreg, wqs, wks):
    blk = 1000
    grid = (_N // blk,)
    row_spec = pl.BlockSpec((blk, _D), lambda i: (i, 0))
    w_spec = pl.BlockSpec((_D, _D), lambda i: (0, 0))
    out = pl.pallas_call(
        _prep_body,
        grid=grid,
        in_specs=[row_spec, pl.BlockSpec((blk, 1), lambda i: (i, 0))]
        + [w_spec] * 11,
        out_specs=[row_spec] * 5,
        out_shape=[jax.ShapeDtypeStruct((_N, _D), _F32)] * 5,
    )(x, tidf, wpoi, wcat, wreg,
      wqs[0], wks[0], wqs[1], wks[1], wqs[2], wks[2], wqs[3], wks[3])
    return out[0], out[1:]


def _edge_body(esp_ref, cnt_ref, ew_ref, we_ref,
               ee0_ref, ee1_ref, ee2_ref, ee3_ref):
    outs = (ee0_ref, ee1_ref, ee2_ref, ee3_ref)
    for t in range(_NT):
        es = esp_ref[t, 0] + esp_ref[t, 1]
        cnt = cnt_ref[t, 0] + cnt_ref[t, 1]
        emb = es / jnp.clip(cnt, 1.0, None)[:, None]
        emb = jnp.dot(emb, we_ref[t].T, preferred_element_type=_F32)
        outs[t][...] = emb * ew_ref[t][:, None]


def _tc_edge(esp, cntp, ew_stack, we_stack):
    return pl.pallas_call(
        _edge_body,
        out_shape=[jax.ShapeDtypeStruct((_NE, _D), _F32)] * 4,
    )(esp, cntp, ew_stack, we_stack)


def _final_body(x_ref, num_ref, den_ref, wv_ref, g_ref, b_ref, o_ref):
    xb = x_ref[...]
    acc = jnp.zeros_like(xb)
    for t in range(_NT):
        nsum = num_ref[t, 0] + num_ref[t, 1]
        dsum = jnp.sum(den_ref[t], axis=0)[:, 0]
        tot = nsum / (dsum + 1e-12)[:, None]
        acc = acc + jnp.dot(tot, wv_ref[t].T, preferred_element_type=_F32)
    out = xb + acc * (1.0 / _NT)
    mean = jnp.mean(out, axis=-1, keepdims=True)
    var = jnp.mean((out - mean) ** 2, axis=-1, keepdims=True)
    o_ref[...] = (out - mean) / jnp.sqrt(var + 1e-5) * g_ref[...] + b_ref[...]


def _tc_final(x, nump, denp4, wv_stack, gamma, beta):
    blk = 1000
    grid = (_N // blk,)
    return pl.pallas_call(
        _final_body,
        grid=grid,
        in_specs=[
            pl.BlockSpec((blk, _D), lambda i: (i, 0)),
            pl.BlockSpec((_NT, _NC, blk, _D), lambda i: (0, 0, i, 0)),
            pl.BlockSpec((_NT, _NC, blk, 1), lambda i: (0, 0, i, 0)),
            pl.BlockSpec((_NT, _D, _D), lambda i: (0, 0, 0)),
            pl.BlockSpec((1, _D), lambda i: (0, 0)),
            pl.BlockSpec((1, _D), lambda i: (0, 0)),
        ],
        out_specs=pl.BlockSpec((blk, _D), lambda i: (i, 0)),
        out_shape=jax.ShapeDtypeStruct((_N, _D), _F32),
    )(x, nump, denp4, wv_stack, gamma, beta)


def _lane_shuffle(v, perm):
    dnums = lax.GatherDimensionNumbers(
        offset_dims=(), collapsed_slice_dims=(0,), start_index_map=(0,))
    return lax.gather(v, perm[:, None], dnums, (1,),
                      mode=lax.GatherScatterMode.PROMISE_IN_BOUNDS)


def _sc_mesh():
    return plsc.VectorSubcoreMesh(core_axis_name="c", subcore_axis_name="s")


def _zero_1d(ref, n):
    def zb(i, carry):
        ref[pl.ds(i * _L, _L)] = jnp.zeros((_L,), _F32)
        return carry
    lax.fori_loop(0, n // _L, zb, 0)


_NB = 4


def _segsum_body(h, nid0, nid1, nid2, nid3, eid0, eid1, eid2, eid3,
                 z2d, es_out, cnt_out,
                 nid_r0, nid_r1, nid_r2, nid_r3,
                 nid_r4, nid_r5, nid_r6, nid_r7,
                 eid_r0, eid_r1, eid_r2, eid_r3,
                 eid_r4, eid_r5, eid_r6, eid_r7,
                 rows_r0, rows_r1, rows_r2, rows_r3,
                 ones_b, zb_b, stage_b, es_acc, cnt_acc,
                 si0, si1, se0, se1, g0, g1, sr0, sr1, sr2, sr3,
                 so0, so1, so2, so3):
    c = lax.axis_index("c")
    s = lax.axis_index("s")
    wid = s * _NC + c
    nids = (nid0, nid1, nid2, nid3)
    eids = (eid0, eid1, eid2, eid3)
    nid_r = (nid_r0, nid_r1, nid_r2, nid_r3,
             nid_r4, nid_r5, nid_r6, nid_r7)
    eid_r = (eid_r0, eid_r1, eid_r2, eid_r3,
             eid_r4, eid_r5, eid_r6, eid_r7)
    rows_r = (rows_r0, rows_r1, rows_r2, rows_r3)
    si_sem = (si0, si1)
    se_sem = (se0, se1)
    g_sem = (g0, g1)
    sr = (sr0, sr1, sr2, sr3)
    so = (so0, so1, so2, so3)
    for j in range(_C // _L):
        ones_b[pl.ds(j * _L, _L)] = jnp.ones((_L,), _F32)
    _zero_1d(zb_b, _NE_PAD)

    def idx_start(t, j, m):
        b, p = m, m % 2
        base = wid * _PER_W + j * _C
        pltpu.async_copy(nids[t].at[pl.ds(base, _C)], nid_r[b], si_sem[p])
        pltpu.async_copy(eids[t].at[pl.ds(base, _C)], eid_r[b], se_sem[p])

    def idx_wait(t, j, m):
        b, p = m, m % 2
        base = wid * _PER_W + j * _C
        pltpu.make_async_copy(nids[t].at[pl.ds(base, _C)], nid_r[b],
                              si_sem[p]).wait()
        pltpu.make_async_copy(eids[t].at[pl.ds(base, _C)], eid_r[b],
                              se_sem[p]).wait()

    def gather_start(m):
        pltpu.async_copy(h.at[nid_r[m]], rows_r[m % _NB], g_sem[m % 2])

    def gather_wait(m):
        pltpu.make_async_copy(h.at[nid_r[m]], rows_r[m % _NB],
                              g_sem[m % 2]).wait()

    def scatter_start(m):
        b4 = m % _NB
        pltpu.async_copy(rows_r[b4], es_acc.at[eid_r[m]], sr[b4], add=True)
        pltpu.async_copy(ones_b, cnt_acc.at[eid_r[m]], so[b4], add=True)

    def scatter_wait(m):
        b4 = m % _NB
        pltpu.make_async_copy(rows_r[b4], es_acc.at[eid_r[m]],
                              sr[b4]).wait()
        pltpu.make_async_copy(ones_b, cnt_acc.at[eid_r[m]],
                              so[b4]).wait()

    for t in range(_NT):
        @pl.when(s == 0)
        def _zero():
            pltpu.sync_copy(z2d.at[pl.ds(0, _NE)], es_acc)
            pltpu.sync_copy(zb_b, cnt_acc)
        plsc.subcore_barrier()

        idx_start(t, 0, 0)
        idx_wait(t, 0, 0)
        gather_start(0)
        idx_start(t, 1, 1)
        n_ss = (_NCH + 3 + _NBI - 1) // _NBI

        def superstep(sidx, carry):
            for b in range(_NBI):
                j = sidx * _NBI + b

                @pl.when(jnp.logical_and(j >= 3, j - 3 < _NCH))
                def _drain():
                    scatter_wait((b + 5) % _NBI)

                @pl.when(j + 1 < _NCH)
                def _pref1():
                    idx_wait(t, j + 1, (b + 1) % _NBI)
                    gather_start((b + 1) % _NBI)

                @pl.when(j + 2 < _NCH)
                def _pref2():
                    idx_start(t, j + 2, (b + 2) % _NBI)

                @pl.when(j < _NCH)
                def _work():
                    gather_wait(b)
                    scatter_start(b)
            return carry
        lax.fori_loop(0, n_ss, superstep, 0)
        plsc.subcore_barrier()

        @pl.when(s == 0)
        def _out():
            pltpu.sync_copy(es_acc, es_out.at[t, c])
            pltpu.sync_copy(cnt_acc.at[pl.ds(0, _NE)], stage_b)
            pltpu.sync_copy(stage_b, cnt_out.at[t, c])
        plsc.subcore_barrier()


def _sc_segsum(h_node, nids, eids, z2d):
    fn = pl.kernel(
        _segsum_body,
        out_type=(jax.ShapeDtypeStruct((_NT, _NC, _NE, _D), _F32),
                  jax.ShapeDtypeStruct((_NT, _NC, _NE), _F32)),
        mesh=_sc_mesh(),
        scratch_types=(
            [pltpu.VMEM((_C,), _I32)] * 16
            + [pltpu.VMEM((_C, _D), _F32)] * 4
            + [pltpu.VMEM((_C,), _F32),
               pltpu.VMEM((_NE_PAD,), _F32),
               pltpu.VMEM((_NE,), _F32),
               pltpu.VMEM_SHARED((_NE, _D), _F32),
               pltpu.VMEM_SHARED((_NE_PAD,), _F32)]
            + [pltpu.SemaphoreType.DMA] * 14
        ),
    )
    return fn(h_node, *nids, *eids, z2d)


_NBI = 8


def _attn_body(xm0, xm1, xm2, xm3, ee0, ee1, ee2, ee3,
               nid0, nid1, nid2, nid3, eid0, eid1, eid2, eid3,
               z2d, num_out, den_out,
               nid_r0, nid_r1, nid_r2, nid_r3,
               nid_r4, nid_r5, nid_r6, nid_r7,
               eid_r0, eid_r1, eid_r2, eid_r3,
               eid_r4, eid_r5, eid_r6, eid_r7,
               xr_r0, xr_r1,
               er_r0, er_r1, er_r2, er_r3,
               ex_r0, ex_r1, ex_r2, ex_r3,
               buf_b, num_acc, den_acc,
               si0, si1, se0, se1, gx0, gx1, ge0, ge1,
               sn0, sn1, sn2, sn3, sd0, sd1, sd2, sd3):
    c = lax.axis_index("c")
    s = lax.axis_index("s")
    wid = s * _NC + c
    xms = (xm0, xm1, xm2, xm3)
    ees = (ee0, ee1, ee2, ee3)
    nids = (nid0, nid1, nid2, nid3)
    eids = (eid0, eid1, eid2, eid3)
    nid_r = (nid_r0, nid_r1, nid_r2, nid_r3,
             nid_r4, nid_r5, nid_r6, nid_r7)
    eid_r = (eid_r0, eid_r1, eid_r2, eid_r3,
             eid_r4, eid_r5, eid_r6, eid_r7)
    xr_r = (xr_r0, xr_r1)
    er_r = (er_r0, er_r1, er_r2, er_r3)
    ex_r = (ex_r0, ex_r1, ex_r2, ex_r3)
    si_sem = (si0, si1)
    se_sem = (se0, se1)
    gx_sem = (gx0, gx1)
    ge_sem = (ge0, ge1)
    sn = (sn0, sn1, sn2, sn3)
    sd = (sd0, sd1, sd2, sd3)
    lanes = lax.iota(_I32, _L)
    lane0 = lanes == 0

    def idx_start(t, j, m):
        b, p = m, m % 2
        base = wid * _PER_W + j * _CB
        pltpu.async_copy(nids[t].at[pl.ds(base, _CB)], nid_r[b], si_sem[p])
        pltpu.async_copy(eids[t].at[pl.ds(base, _CB)], eid_r[b], se_sem[p])

    def idx_wait(t, j, m):
        b, p = m, m % 2
        base = wid * _PER_W + j * _CB
        pltpu.make_async_copy(nids[t].at[pl.ds(base, _CB)], nid_r[b],
                              si_sem[p]).wait()
        pltpu.make_async_copy(eids[t].at[pl.ds(base, _CB)], eid_r[b],
                              se_sem[p]).wait()

    def gather_start(t, m):
        b, b4, b2 = m, m % _NB, m % 2
        pltpu.async_copy(xms[t].at[nid_r[b]], xr_r[b2], gx_sem[b2])
        pltpu.async_copy(ees[t].at[eid_r[b]], er_r[b4], ge_sem[b2])

    def gather_wait(t, m):
        b, b4, b2 = m, m % _NB, m % 2
        pltpu.make_async_copy(xms[t].at[nid_r[b]], xr_r[b2],
                              gx_sem[b2]).wait()
        pltpu.make_async_copy(ees[t].at[eid_r[b]], er_r[b4],
                              ge_sem[b2]).wait()

    def scatter_start(m):
        b, b4 = m, m % _NB
        pltpu.async_copy(er_r[b4], num_acc.at[nid_r[b]], sn[b4], add=True)
        pltpu.async_copy(ex_r[b4].at[pl.ds(0, _CB)], den_acc.at[nid_r[b]],
                         sd[b4], add=True)

    def scatter_wait(m):
        b, b4 = m, m % _NB
        pltpu.make_async_copy(er_r[b4], num_acc.at[nid_r[b]],
                              sn[b4]).wait()
        pltpu.make_async_copy(ex_r[b4].at[pl.ds(0, _CB)],
                              den_acc.at[nid_r[b]], sd[b4]).wait()

    def compute(m):
        xr_b = xr_r[m % 2]
        er_b = er_r[m % _NB]
        ex_b = ex_r[m % _NB]

        def inc(i, icarry):
            evs = []
            acc = jnp.zeros((_L,), _F32)
            for d in range(_D // _L):
                xv = xr_b[i, pl.ds(d * _L, _L)]
                ev = er_b[i, pl.ds(d * _L, _L)]
                evs.append(ev)
                acc = acc + xv * ev
            sc = acc
            for k in (8, 4, 2, 1):
                perm = jnp.bitwise_xor(lanes, k)
                sc = sc + _lane_shuffle(sc, perm)
            exv = jnp.exp(sc)
            for d in range(_D // _L):
                er_b[i, pl.ds(d * _L, _L)] = evs[d] * exv
            ex_b[pl.ds(i, _L)] = exv
            return icarry
        lax.fori_loop(0, _CB, inc, 0)

    for t in range(_NT):
        @pl.when(s == 0)
        def _zero():
            _zero_1d(buf_b, _N - _L)
            buf_b[pl.ds(_N - _L, _L)] = jnp.zeros((_L,), _F32)
            pltpu.sync_copy(z2d, num_acc)
            pltpu.sync_copy(buf_b, den_acc)
        plsc.subcore_barrier()

        idx_start(t, 0, 0)
        idx_wait(t, 0, 0)
        gather_start(t, 0)
        idx_start(t, 1, 1)
        n_ss = (_NCHB + 3 + _NBI - 1) // _NBI

        def superstep(sidx, carry):
            for b in range(_NBI):
                j = sidx * _NBI + b

                @pl.when(jnp.logical_and(j >= 3, j - 3 < _NCHB))
                def _drain():
                    scatter_wait((b + 5) % _NBI)

                @pl.when(j + 1 < _NCHB)
                def _pref1():
                    idx_wait(t, j + 1, (b + 1) % _NBI)
                    gather_start(t, (b + 1) % _NBI)

                @pl.when(j + 2 < _NCHB)
                def _pref2():
                    idx_start(t, j + 2, (b + 2) % _NBI)

                @pl.when(j < _NCHB)
                def _work():
                    gather_wait(t, b)
                    compute(b)
                    scatter_start(b)
            return carry
        lax.fori_loop(0, n_ss, superstep, 0)
        plsc.subcore_barrier()

        @pl.when(s == 0)
        def _out():
            pltpu.sync_copy(num_acc, num_out.at[t, c])
            pltpu.sync_copy(den_acc, buf_b)
            pltpu.sync_copy(buf_b, den_out.at[t, c])
        plsc.subcore_barrier()


def _sc_attn(xms, ees, nids, eids, z2d):
    fn = pl.kernel(
        _attn_body,
        out_type=(jax.ShapeDtypeStruct((_NT, _NC, _N, _D), _F32),
                  jax.ShapeDtypeStruct((_NT, _NC, _N), _F32)),
        mesh=_sc_mesh(),
        scratch_types=(
            [pltpu.VMEM((_CB,), _I32)] * 16
            + [pltpu.VMEM((_CB, _D), _F32)] * 6
            + [pltpu.VMEM((_CB + _L,), _F32)] * 4
            + [pltpu.VMEM((_N,), _F32),
               pltpu.VMEM_SHARED((_N, _D), _F32),
               pltpu.VMEM_SHARED((_N,), _F32)]
            + [pltpu.SemaphoreType.DMA] * 16
        ),
    )
    return fn(*xms, *ees, *nids, *eids, z2d)


def kernel(x, node_type_ids,
           node_ids_func, edge_ids_func, edge_weight_func,
           node_ids_region, edge_ids_region, edge_weight_region,
           node_ids_geo, edge_ids_geo, edge_weight_geo,
           node_ids_mob, edge_ids_mob, edge_weight_mob,
           W_poi, W_cat, W_reg,
           W_edge_func, Wq_func, Wk_func, Wv_func,
           W_edge_region, Wq_region, Wk_region, Wv_region,
           W_edge_geo, Wq_geo, Wk_geo, Wv_geo,
           W_edge_mob, Wq_mob, Wk_mob, Wv_mob,
           ln_gamma, ln_beta):
    nids = (node_ids_func, node_ids_region, node_ids_geo, node_ids_mob)
    eids = (edge_ids_func, edge_ids_region, edge_ids_geo, edge_ids_mob)
    ews = (edge_weight_func, edge_weight_region, edge_weight_geo,
           edge_weight_mob)
    wqs = (Wq_func, Wq_region, Wq_geo, Wq_mob)
    wks = (Wk_func, Wk_region, Wk_geo, Wk_mob)
    wvs = (Wv_func, Wv_region, Wv_geo, Wv_mob)
    wes = (W_edge_func, W_edge_region, W_edge_geo, W_edge_mob)

    tidf = node_type_ids.astype(_F32).reshape(_N, 1)
    z2d = jnp.zeros((_N, _D), _F32)

    h_node, xms = _tc_prep(x, tidf, W_poi, W_cat, W_reg, wqs, wks)
    esp, cntp = _sc_segsum(h_node, nids, eids, z2d)
    ees = _tc_edge(esp, cntp, jnp.stack(ews), jnp.stack(wes))
    nump, denp = _sc_attn(xms, ees, nids, eids, z2d)
    out = _tc_final(x, nump, denp.reshape(_NT, _NC, _N, 1),
                    jnp.stack(wvs), ln_gamma.reshape(1, _D),
                    ln_beta.reshape(1, _D))
    return out


# speedup vs baseline: 1.0945x; 1.0001x over previous
import math

import jax
import jax.numpy as jnp
from jax import lax
from jax.experimental import pallas as pl
from jax.experimental.pallas import tpu as pltpu
from jax.experimental.pallas import tpu_sc as plsc

_N = 10000
_D = 128
_E = 320000
_NE = 5000
_NT = 4

_NC = 2
_NS = 16
_NW = _NC * _NS
_L = 16

_NE_PAD = 5120
_N_PAD = 10112
_C = 80
_PER_W = _E // _NW
_NCH = _PER_W // _C
_CB = 40
_NCHB = _PER_W // _CB

_F32 = jnp.float32
_I32 = jnp.int32


def _prep_body(x_ref, tid_ref, wpoi_ref, wcat_ref, wreg_ref,
               wq0, wk0, wq1, wk1, wq2, wk2, wq3, wk3,
               h_ref, xm0_ref, xm1_ref, xm2_ref, xm3_ref):
    xb = x_ref[...]
    hp = jnp.dot(xb, wpoi_ref[...].T, preferred_element_type=_F32)
    hc = jnp.dot(xb, wcat_ref[...].T, preferred_element_type=_F32)
    hr = jnp.dot(xb, wreg_ref[...].T, preferred_element_type=_F32)
    tid = tid_ref[...]
    h_ref[...] = jnp.where(tid == 0.0, hp, jnp.where(tid == 1.0, hc, hr))
    scale = 1.0 / math.sqrt(_D)
    for wq, wk, xm in ((wq0, wk0, xm0_ref), (wq1, wk1, xm1_ref),
                       (wq2, wk2, xm2_ref), (wq3, wk3, xm3_ref)):
        m = jnp.dot(wq[...].T, wk[...], preferred_element_type=_F32)
        xm[...] = jnp.dot(xb, m, preferred_element_type=_F32) * scale


def _tc_prep(x, tidf, wpoi, wcat, wreg, wqs, wks):
    blk = 1000
    grid = (_N // blk,)
    row_spec = pl.BlockSpec((blk, _D), lambda i: (i, 0))
    w_spec = pl.BlockSpec((_D, _D), lambda i: (0, 0))
    out = pl.pallas_call(
        _prep_body,
        grid=grid,
        in_specs=[row_spec, pl.BlockSpec((blk, 1), lambda i: (i, 0))]
        + [w_spec] * 11,
        out_specs=[row_spec] * 5,
        out_shape=[jax.ShapeDtypeStruct((_N, _D), _F32)] * 5,
    )(x, tidf, wpoi, wcat, wreg,
      wqs[0], wks[0], wqs[1], wks[1], wqs[2], wks[2], wqs[3], wks[3])
    return out[0], out[1:]


def _edge_body(esp_ref, cnt_ref, ew_ref, we_ref,
               ee0_ref, ee1_ref, ee2_ref, ee3_ref):
    outs = (ee0_ref, ee1_ref, ee2_ref, ee3_ref)
    for t in range(_NT):
        es = esp_ref[t, 0] + esp_ref[t, 1]
        cnt = cnt_ref[t, 0] + cnt_ref[t, 1]
        emb = es / jnp.clip(cnt, 1.0, None)[:, None]
        emb = jnp.dot(emb, we_ref[t].T, preferred_element_type=_F32)
        outs[t][...] = emb * ew_ref[t][:, None]


def _tc_edge(esp, cntp, ew_stack, we_stack):
    return pl.pallas_call(
        _edge_body,
        out_shape=[jax.ShapeDtypeStruct((_NE, _D), _F32)] * 4,
    )(esp, cntp, ew_stack, we_stack)


def _final_body(x_ref, num_ref, den_ref, wv_ref, g_ref, b_ref, o_ref):
    xb = x_ref[...]
    acc = jnp.zeros_like(xb)
    for t in range(_NT):
        nsum = num_ref[t, 0] + num_ref[t, 1]
        dsum = jnp.sum(den_ref[t], axis=0)[:, 0]
        tot = nsum / (dsum + 1e-12)[:, None]
        acc = acc + jnp.dot(tot, wv_ref[t].T, preferred_element_type=_F32)
    out = xb + acc * (1.0 / _NT)
    mean = jnp.mean(out, axis=-1, keepdims=True)
    var = jnp.mean((out - mean) ** 2, axis=-1, keepdims=True)
    o_ref[...] = (out - mean) / jnp.sqrt(var + 1e-5) * g_ref[...] + b_ref[...]


def _tc_final(x, nump, denp4, wv_stack, gamma, beta):
    blk = 1000
    grid = (_N // blk,)
    return pl.pallas_call(
        _final_body,
        grid=grid,
        in_specs=[
            pl.BlockSpec((blk, _D), lambda i: (i, 0)),
            pl.BlockSpec((_NT, _NC, blk, _D), lambda i: (0, 0, i, 0)),
            pl.BlockSpec((_NT, _NC, blk, 1), lambda i: (0, 0, i, 0)),
            pl.BlockSpec((_NT, _D, _D), lambda i: (0, 0, 0)),
            pl.BlockSpec((1, _D), lambda i: (0, 0)),
            pl.BlockSpec((1, _D), lambda i: (0, 0)),
        ],
        out_specs=pl.BlockSpec((blk, _D), lambda i: (i, 0)),
        out_shape=jax.ShapeDtypeStruct((_N, _D), _F32),
    )(x, nump, denp4, wv_stack, gamma, beta)


def _lane_shuffle(v, perm):
    dnums = lax.GatherDimensionNumbers(
        offset_dims=(), collapsed_slice_dims=(0,), start_index_map=(0,))
    return lax.gather(v, perm[:, None], dnums, (1,),
                      mode=lax.GatherScatterMode.PROMISE_IN_BOUNDS)


def _sc_mesh():
    return plsc.VectorSubcoreMesh(core_axis_name="c", subcore_axis_name="s")


def _zero_1d(ref, n):
    def zb(i, carry):
        ref[pl.ds(i * _L, _L)] = jnp.zeros((_L,), _F32)
        return carry
    lax.fori_loop(0, n // _L, zb, 0)


_NB = 4


def _segsum_body(h, nid0, nid1, nid2, nid3, eid0, eid1, eid2, eid3,
                 z2d, es_out, cnt_out,
                 nid_r0, nid_r1, nid_r2, nid_r3,
                 nid_r4, nid_r5, nid_r6, nid_r7,
                 eid_r0, eid_r1, eid_r2, eid_r3,
                 eid_r4, eid_r5, eid_r6, eid_r7,
                 rows_r0, rows_r1, rows_r2, rows_r3,
                 ones_b, zb_b, stage_b, es_acc, cnt_acc,
                 si0, si1, se0, se1, g0, g1, sr0, sr1, sr2, sr3,
                 so0, so1, so2, so3):
    c = lax.axis_index("c")
    s = lax.axis_index("s")
    wid = s * _NC + c
    nids = (nid0, nid1, nid2, nid3)
    eids = (eid0, eid1, eid2, eid3)
    nid_r = (nid_r0, nid_r1, nid_r2, nid_r3,
             nid_r4, nid_r5, nid_r6, nid_r7)
    eid_r = (eid_r0, eid_r1, eid_r2, eid_r3,
             eid_r4, eid_r5, eid_r6, eid_r7)
    rows_r = (rows_r0, rows_r1, rows_r2, rows_r3)
    si_sem = (si0, si1)
    se_sem = (se0, se1)
    g_sem = (g0, g1)
    sr = (sr0, sr1, sr2, sr3)
    so = (so0, so1, so2, so3)
    for j in range(_C // _L):
        ones_b[pl.ds(j * _L, _L)] = jnp.ones((_L,), _F32)
    _zero_1d(zb_b, _NE_PAD)

    def idx_start(t, j, m):
        b, p = m, m % 2
        base = wid * _PER_W + j * _C
        pltpu.async_copy(nids[t].at[pl.ds(base, _C)], nid_r[b], si_sem[p])
        pltpu.async_copy(eids[t].at[pl.ds(base, _C)], eid_r[b], se_sem[p])

    def idx_wait(t, j, m):
        b, p = m, m % 2
        base = wid * _PER_W + j * _C
        pltpu.make_async_copy(nids[t].at[pl.ds(base, _C)], nid_r[b],
                              si_sem[p]).wait()
        pltpu.make_async_copy(eids[t].at[pl.ds(base, _C)], eid_r[b],
                              se_sem[p]).wait()

    def gather_start(m):
        pltpu.async_copy(h.at[nid_r[m]], rows_r[m % _NB], g_sem[m % 2])

    def gather_wait(m):
        pltpu.make_async_copy(h.at[nid_r[m]], rows_r[m % _NB],
                              g_sem[m % 2]).wait()

    def scatter_start(m):
        b4 = m % _NB
        pltpu.async_copy(rows_r[b4], es_acc.at[eid_r[m]], sr[b4], add=True)
        pltpu.async_copy(ones_b, cnt_acc.at[eid_r[m]], so[b4], add=True)

    def scatter_wait(m):
        b4 = m % _NB
        pltpu.make_async_copy(rows_r[b4], es_acc.at[eid_r[m]],
                              sr[b4]).wait()
        pltpu.make_async_copy(ones_b, cnt_acc.at[eid_r[m]],
                              so[b4]).wait()

    for t in range(_NT):
        @pl.when(s == 0)
        def _zero():
            pltpu.sync_copy(z2d.at[pl.ds(0, _NE)], es_acc)
            pltpu.sync_copy(zb_b, cnt_acc)
        plsc.subcore_barrier()

        idx_start(t, 0, 0)
        idx_wait(t, 0, 0)
        gather_start(0)
        idx_start(t, 1, 1)
        n_ss = (_NCH + 3 + _NBI - 1) // _NBI

        def superstep(sidx, carry):
            for b in range(_NBI):
                j = sidx * _NBI + b

                @pl.when(jnp.logical_and(j >= 3, j - 3 < _NCH))
                def _drain():
                    scatter_wait((b + 5) % _NBI)

                @pl.when(j + 1 < _NCH)
                def _pref1():
                    idx_wait(t, j + 1, (b + 1) % _NBI)
                    gather_start((b + 1) % _NBI)

                @pl.when(j + 2 < _NCH)
                def _pref2():
                    idx_start(t, j + 2, (b + 2) % _NBI)

                @pl.when(j < _NCH)
                def _work():
                    gather_wait(b)
                    scatter_start(b)
            return carry
        lax.fori_loop(0, n_ss, superstep, 0)
        plsc.subcore_barrier()

        @pl.when(s == 0)
        def _out():
            pltpu.sync_copy(es_acc, es_out.at[t, c])
            pltpu.sync_copy(cnt_acc.at[pl.ds(0, _NE)], stage_b)
            pltpu.sync_copy(stage_b, cnt_out.at[t, c])
        plsc.subcore_barrier()


def _sc_segsum(h_node, nids, eids, z2d):
    fn = pl.kernel(
        _segsum_body,
        out_type=(jax.ShapeDtypeStruct((_NT, _NC, _NE, _D), _F32),
                  jax.ShapeDtypeStruct((_NT, _NC, _NE), _F32)),
        mesh=_sc_mesh(),
        scratch_types=(
            [pltpu.VMEM((_C,), _I32)] * 16
            + [pltpu.VMEM((_C, _D), _F32)] * 4
            + [pltpu.VMEM((_C,), _F32),
               pltpu.VMEM((_NE_PAD,), _F32),
               pltpu.VMEM((_NE,), _F32),
               pltpu.VMEM_SHARED((_NE, _D), _F32),
               pltpu.VMEM_SHARED((_NE_PAD,), _F32)]
            + [pltpu.SemaphoreType.DMA] * 14
        ),
    )
    return fn(h_node, *nids, *eids, z2d)


_NBI = 8


def _attn_body(xm0, xm1, xm2, xm3, ee0, ee1, ee2, ee3,
               nid0, nid1, nid2, nid3, eid0, eid1, eid2, eid3,
               z2d, num_out, den_out,
               nid_r0, nid_r1, nid_r2, nid_r3,
               nid_r4, nid_r5, nid_r6, nid_r7,
               eid_r0, eid_r1, eid_r2, eid_r3,
               eid_r4, eid_r5, eid_r6, eid_r7,
               xr_r0, xr_r1,
               er_r0, er_r1, er_r2, er_r3,
               ex_r0, ex_r1, ex_r2, ex_r3,
               buf_b, num_acc, den_acc,
               si0, si1, se0, se1, gx0, gx1, ge0, ge1,
               sn0, sn1, sn2, sn3, sd0, sd1, sd2, sd3):
    c = lax.axis_index("c")
    s = lax.axis_index("s")
    wid = s * _NC + c
    xms = (xm0, xm1, xm2, xm3)
    ees = (ee0, ee1, ee2, ee3)
    nids = (nid0, nid1, nid2, nid3)
    eids = (eid0, eid1, eid2, eid3)
    nid_r = (nid_r0, nid_r1, nid_r2, nid_r3,
             nid_r4, nid_r5, nid_r6, nid_r7)
    eid_r = (eid_r0, eid_r1, eid_r2, eid_r3,
             eid_r4, eid_r5, eid_r6, eid_r7)
    xr_r = (xr_r0, xr_r1)
    er_r = (er_r0, er_r1, er_r2, er_r3)
    ex_r = (ex_r0, ex_r1, ex_r2, ex_r3)
    si_sem = (si0, si1)
    se_sem = (se0, se1)
    gx_sem = (gx0, gx1)
    ge_sem = (ge0, ge1)
    sn = (sn0, sn1, sn2, sn3)
    sd = (sd0, sd1, sd2, sd3)
    lanes = lax.iota(_I32, _L)

    def idx_start(t, j, m):
        b, p = m, m % 2
        base = wid * _PER_W + j * _CB
        pltpu.async_copy(nids[t].at[pl.ds(base, _CB)], nid_r[b], si_sem[p])
        pltpu.async_copy(eids[t].at[pl.ds(base, _CB)], eid_r[b], se_sem[p])

    def idx_wait(t, j, m):
        b, p = m, m % 2
        base = wid * _PER_W + j * _CB
        pltpu.make_async_copy(nids[t].at[pl.ds(base, _CB)], nid_r[b],
                              si_sem[p]).wait()
        pltpu.make_async_copy(eids[t].at[pl.ds(base, _CB)], eid_r[b],
                              se_sem[p]).wait()

    def gather_start(t, m):
        b, b4, b2 = m, m % _NB, m % 2
        pltpu.async_copy(xms[t].at[nid_r[b]], xr_r[b2], gx_sem[b2])
        pltpu.async_copy(ees[t].at[eid_r[b]], er_r[b4], ge_sem[b2])

    def gather_wait(t, m):
        b, b4, b2 = m, m % _NB, m % 2
        pltpu.make_async_copy(xms[t].at[nid_r[b]], xr_r[b2],
                              gx_sem[b2]).wait()
        pltpu.make_async_copy(ees[t].at[eid_r[b]], er_r[b4],
                              ge_sem[b2]).wait()

    def scatter_start(m):
        b, b4 = m, m % _NB
        pltpu.async_copy(er_r[b4], num_acc.at[nid_r[b]], sn[b4], add=True)
        pltpu.async_copy(ex_r[b4].at[pl.ds(0, _CB)], den_acc.at[nid_r[b]],
                         sd[b4], add=True)

    def scatter_wait(m):
        b, b4 = m, m % _NB
        pltpu.make_async_copy(er_r[b4], num_acc.at[nid_r[b]],
                              sn[b4]).wait()
        pltpu.make_async_copy(ex_r[b4].at[pl.ds(0, _CB)],
                              den_acc.at[nid_r[b]], sd[b4]).wait()

    def compute(m):
        xr_b = xr_r[m % 2]
        er_b = er_r[m % _NB]
        ex_b = ex_r[m % _NB]

        def inc(i, icarry):
            evs = []
            acc = jnp.zeros((_L,), _F32)
            for d in range(_D // _L):
                xv = xr_b[i, pl.ds(d * _L, _L)]
                ev = er_b[i, pl.ds(d * _L, _L)]
                evs.append(ev)
                acc = acc + xv * ev
            sc = acc
            for k in (8, 4, 2, 1):
                perm = jnp.bitwise_xor(lanes, k)
                sc = sc + _lane_shuffle(sc, perm)
            exv = jnp.exp(sc)
            for d in range(_D // _L):
                er_b[i, pl.ds(d * _L, _L)] = evs[d] * exv
            ex_b[pl.ds(i, _L)] = exv
            return icarry
        lax.fori_loop(0, _CB, inc, 0)

    for t in range(_NT):
        @pl.when(s == 0)
        def _zero():
            _zero_1d(buf_b, _N - _L)
            buf_b[pl.ds(_N - _L, _L)] = jnp.zeros((_L,), _F32)
            pltpu.sync_copy(z2d, num_acc)
            pltpu.sync_copy(buf_b, den_acc)
        plsc.subcore_barrier()

        idx_start(t, 0, 0)
        idx_wait(t, 0, 0)
        gather_start(t, 0)
        idx_start(t, 1, 1)
        n_ss = (_NCHB + 3 + _NBI - 1) // _NBI

        def superstep(sidx, carry):
            for b in range(_NBI):
                j = sidx * _NBI + b

                @pl.when(jnp.logical_and(j >= 3, j - 3 < _NCHB))
                def _drain():
                    scatter_wait((b + 5) % _NBI)

                @pl.when(j + 1 < _NCHB)
                def _pref1():
                    idx_wait(t, j + 1, (b + 1) % _NBI)
                    gather_start(t, (b + 1) % _NBI)

                @pl.when(j + 2 < _NCHB)
                def _pref2():
                    idx_start(t, j + 2, (b + 2) % _NBI)

                @pl.when(j < _NCHB)
                def _work():
                    gather_wait(t, b)
                    compute(b)
                    scatter_start(b)
            return carry
        lax.fori_loop(0, n_ss, superstep, 0)
        plsc.subcore_barrier()

        @pl.when(s == 0)
        def _out():
            pltpu.sync_copy(num_acc, num_out.at[t, c])
            pltpu.sync_copy(den_acc, buf_b)
            pltpu.sync_copy(buf_b, den_out.at[t, c])
        plsc.subcore_barrier()


def _sc_attn(xms, ees, nids, eids, z2d):
    fn = pl.kernel(
        _attn_body,
        out_type=(jax.ShapeDtypeStruct((_NT, _NC, _N, _D), _F32),
                  jax.ShapeDtypeStruct((_NT, _NC, _N), _F32)),
        mesh=_sc_mesh(),
        scratch_types=(
            [pltpu.VMEM((_CB,), _I32)] * 16
            + [pltpu.VMEM((_CB, _D), _F32)] * 6
            + [pltpu.VMEM((_CB + _L,), _F32)] * 4
            + [pltpu.VMEM((_N,), _F32),
               pltpu.VMEM_SHARED((_N, _D), _F32),
               pltpu.VMEM_SHARED((_N,), _F32)]
            + [pltpu.SemaphoreType.DMA] * 16
        ),
    )
    return fn(*xms, *ees, *nids, *eids, z2d)


def kernel(x, node_type_ids,
           node_ids_func, edge_ids_func, edge_weight_func,
           node_ids_region, edge_ids_region, edge_weight_region,
           node_ids_geo, edge_ids_geo, edge_weight_geo,
           node_ids_mob, edge_ids_mob, edge_weight_mob,
           W_poi, W_cat, W_reg,
           W_edge_func, Wq_func, Wk_func, Wv_func,
           W_edge_region, Wq_region, Wk_region, Wv_region,
           W_edge_geo, Wq_geo, Wk_geo, Wv_geo,
           W_edge_mob, Wq_mob, Wk_mob, Wv_mob,
           ln_gamma, ln_beta):
    nids = (node_ids_func, node_ids_region, node_ids_geo, node_ids_mob)
    eids = (edge_ids_func, edge_ids_region, edge_ids_geo, edge_ids_mob)
    ews = (edge_weight_func, edge_weight_region, edge_weight_geo,
           edge_weight_mob)
    wqs = (Wq_func, Wq_region, Wq_geo, Wq_mob)
    wks = (Wk_func, Wk_region, Wk_geo, Wk_mob)
    wvs = (Wv_func, Wv_region, Wv_geo, Wv_mob)
    wes = (W_edge_func, W_edge_region, W_edge_geo, W_edge_mob)

    tidf = node_type_ids.astype(_F32).reshape(_N, 1)
    z2d = jnp.zeros((_N, _D), _F32)

    h_node, xms = _tc_prep(x, tidf, W_poi, W_cat, W_reg, wqs, wks)
    esp, cntp = _sc_segsum(h_node, nids, eids, z2d)
    ees = _tc_edge(esp, cntp, jnp.stack(ews), jnp.stack(wes))
    nump, denp = _sc_attn(xms, ees, nids, eids, z2d)
    out = _tc_final(x, nump, denp.reshape(_NT, _NC, _N, 1),
                    jnp.stack(wvs), ln_gamma.reshape(1, _D),
                    ln_beta.reshape(1, _D))
    return out
